# baseline (speedup 1.0000x reference)
import functools

import jax
import jax.numpy as jnp
from jax import lax
from jax.experimental import pallas as pl
from jax.experimental.pallas import tpu as pltpu
from jax.experimental.pallas import tpu_sc as plsc

NH = 8
HD = 128
K = 64
SBLK = 1024
LANES = 16


def _gate_kernel(hs_ref, wg_ref, o_ref):
    o_ref[0] = jax.lax.dot_general(
        wg_ref[...], hs_ref[0], (((1,), (1,)), ((), ())),
        preferred_element_type=jnp.float32)


def _topk_sc_kernel(gates_hbm, hs2_hbm, part_hbm, gath_hbm, g_v, bmax_v,
                    idx_v, maskl_v, rows_a, rows_b, sem_a, sem_b):
    s = gates_hbm.shape[2]
    nb = s // LANES
    c = lax.axis_index("c")
    sid = lax.axis_index("s")
    b = c
    h = sid % NH
    half = sid // NH
    iota = lax.iota(jnp.int32, LANES)
    zeros = jnp.zeros((LANES,), jnp.float32)
    lane0 = iota == 0

    def store1(ref, pos, val):
        plsc.store_scatter(ref, [jnp.full((LANES,), pos, jnp.int32)],
                           jnp.full((LANES,), val), mask=lane0)

    def zero_body(j, _):
        maskl_v[pl.ds(j * LANES, LANES)] = zeros
        return 0
    lax.fori_loop(0, nb, zero_body, 0)

    pltpu.sync_copy(gates_hbm.at[b, h], g_v)

    def bmax_body(j, _):
        store1(bmax_v, j, jnp.max(g_v[pl.ds(j * LANES, LANES)]))
        return 0
    lax.fori_loop(0, nb, bmax_body, 0)

    def extract_body(i, _):
        vs = [bmax_v[pl.ds(t * LANES, LANES)] for t in range(nb // LANES)]
        acc = vs[0]
        for t in range(1, len(vs)):
            acc = jnp.maximum(acc, vs[t])
        m = jnp.max(acc)
        cand = jnp.full((LANES,), nb, jnp.int32)
        for t in range(len(vs)):
            cand = jnp.minimum(
                cand, jnp.where(vs[t] == m, iota + t * LANES, nb))
        bb = jnp.min(cand)
        v = g_v[pl.ds(bb * LANES, LANES)]
        lane = jnp.min(jnp.where(v == m, iota, LANES))
        gidx = bb * LANES + lane
        store1(idx_v, i, gidx)
        store1(maskl_v, gidx, 1.0)
        v2 = jnp.where(iota == lane, -jnp.inf, v)
        g_v[pl.ds(bb * LANES, LANES)] = v2
        store1(bmax_v, bb, jnp.max(v2))
        return 0
    lax.fori_loop(0, K, extract_body, 0)

    @pl.when(half == 0)
    def _():
        pltpu.sync_copy(maskl_v, part_hbm.at[b, h])

    ch = K // 4
    bufs = [rows_a, rows_b]
    sems = [sem_a, sem_b]

    def start(c2):
        gv = idx_v[pl.ds((half * 2 + c2) * ch, ch)] + b * s
        return pltpu.async_copy(hs2_hbm.at[gv], bufs[c2 % 2], sems[c2 % 2])

    cp = start(0)
    for c2 in range(2):
        nxt = start(c2 + 1) if c2 + 1 < 2 else None
        cp.wait()
        pltpu.sync_copy(
            bufs[c2 % 2],
            gath_hbm.at[b, h, pl.ds((half * 2 + c2) * ch, ch)])
        cp = nxt


def _proj_kernel(g_ref, wp_ref, part_ref, o_ref, mask_ref):
    o_ref[0] = jax.lax.dot_general(
        g_ref[0, 0], wp_ref[0], (((1,), (1,)), ((), ())),
        preferred_element_type=jnp.float32)
    mask_ref[0] = jnp.max(part_ref[0], axis=0, keepdims=True)


def kernel(hidden_states, W_proj, W_gate):
    bfull, s, hidden = hidden_states.shape
    bh = bfull // 2
    wp3 = W_proj.reshape(NH, HD, hidden)

    outs = []
    masks = []
    for p in range(2):
        hsp = hidden_states[p * bh:(p + 1) * bh]

        gates = pl.pallas_call(
            _gate_kernel,
            grid=(bh, s // SBLK),
            in_specs=[
                pl.BlockSpec((1, SBLK, hidden), lambda i, j: (i, j, 0)),
                pl.BlockSpec((NH, hidden), lambda i, j: (0, 0)),
            ],
            out_specs=pl.BlockSpec((1, NH, SBLK), lambda i, j: (i, 0, j)),
            out_shape=jax.ShapeDtypeStruct((bh, NH, s), jnp.float32),
        )(hsp, W_gate)

        part, gath = pl.kernel(
            _topk_sc_kernel,
            out_type=[
                jax.ShapeDtypeStruct((bh, NH, s), jnp.float32),
                jax.ShapeDtypeStruct((bh, NH, K, hidden), jnp.float32),
            ],
            mesh=plsc.VectorSubcoreMesh(
                core_axis_name="c", subcore_axis_name="s"),
            compiler_params=pltpu.CompilerParams(needs_layout_passes=False),
            scratch_types=[
                pltpu.VMEM((s,), jnp.float32),
                pltpu.VMEM((s // LANES,), jnp.float32),
                pltpu.VMEM((K,), jnp.int32),
                pltpu.VMEM((s,), jnp.float32),
                pltpu.VMEM((K // 4, hidden), jnp.float32),
                pltpu.VMEM((K // 4, hidden), jnp.float32),
                pltpu.SemaphoreType.DMA,
                pltpu.SemaphoreType.DMA,
            ],
        )(gates, hsp.reshape(bh * s, hidden))

        out_p, mask_p = pl.pallas_call(
            _proj_kernel,
            grid=(NH, bh),
            in_specs=[
                pl.BlockSpec((1, 1, K, hidden), lambda j, i: (i, j, 0, 0)),
                pl.BlockSpec((1, HD, hidden), lambda j, i: (j, 0, 0)),
                pl.BlockSpec((1, NH, s), lambda j, i: (i, 0, 0)),
            ],
            out_specs=[
                pl.BlockSpec((1, K, HD), lambda j, i: (i, j, 0)),
                pl.BlockSpec((1, 1, s), lambda j, i: (i, 0, 0)),
            ],
            out_shape=[
                jax.ShapeDtypeStruct((bh, NH * K, HD), jnp.float32),
                jax.ShapeDtypeStruct((bh, 1, s), jnp.float32),
            ],
        )(gath, wp3, part)
        outs.append(out_p)
        masks.append(mask_p)

    out_states = jnp.concatenate(outs, axis=0)
    maskf = jnp.concatenate(masks, axis=0)
    return out_states, maskf.reshape(bfull, s).astype(bool)


# speedup vs baseline: 1.7571x; 1.7571x over previous
import functools

import jax
import jax.numpy as jnp
from jax import lax
from jax.experimental import pallas as pl
from jax.experimental.pallas import tpu as pltpu
from jax.experimental.pallas import tpu_sc as plsc

NH = 8
HD = 128
K = 64
SBLK = 1024
LANES = 16


def _gate_kernel(hs_ref, wg_ref, o_ref):
    o_ref[0] = jax.lax.dot_general(
        wg_ref[...], hs_ref[0], (((1,), (1,)), ((), ())),
        preferred_element_type=jnp.float32)


def _topk_sc_kernel(gates_hbm, hs2_hbm, part_hbm, gath_hbm, g_v, bmax_v,
                    idx_v, maskl_v, rows_a, rows_b, semr_a, semr_b,
                    semw_a, semw_b):
    s = gates_hbm.shape[2]
    nb = s // LANES
    c = lax.axis_index("c")
    sid = lax.axis_index("s")
    b = c * 2 + sid // NH
    h = sid % NH
    iota = lax.iota(jnp.int32, LANES)
    zeros = jnp.zeros((LANES,), jnp.float32)
    lane0 = iota == 0

    def store1(ref, pos, val):
        plsc.store_scatter(ref, [jnp.full((LANES,), pos, jnp.int32)],
                           jnp.full((LANES,), val), mask=lane0)

    def zero_body(j, _):
        maskl_v[pl.ds(j * LANES, LANES)] = zeros
        return 0
    lax.fori_loop(0, nb, zero_body, 0)

    pltpu.sync_copy(gates_hbm.at[b, h], g_v)

    def bmax_body(j, _):
        store1(bmax_v, j, jnp.max(g_v[pl.ds(j * LANES, LANES)]))
        return 0
    lax.fori_loop(0, nb, bmax_body, 0)

    def extract_body(i, _):
        vs = [bmax_v[pl.ds(t * LANES, LANES)] for t in range(nb // LANES)]
        acc = vs[0]
        for t in range(1, len(vs)):
            acc = jnp.maximum(acc, vs[t])
        m = jnp.max(acc)
        cand = jnp.full((LANES,), nb, jnp.int32)
        for t in range(len(vs)):
            cand = jnp.minimum(
                cand, jnp.where(vs[t] == m, iota + t * LANES, nb))
        bb = jnp.min(cand)
        v = g_v[pl.ds(bb * LANES, LANES)]
        lane = jnp.min(jnp.where(v == m, iota, LANES))
        gidx = bb * LANES + lane
        store1(idx_v, i, gidx)
        store1(maskl_v, gidx, 1.0)
        v2 = jnp.where(iota == lane, -jnp.inf, v)
        g_v[pl.ds(bb * LANES, LANES)] = v2
        store1(bmax_v, bb, jnp.max(v2))
        return 0
    ch = K // 4
    bufs = [rows_a, rows_b]
    semr = [semr_a, semr_b]
    semw = [semw_a, semw_b]
    rd = {}
    wr = {}
    for r in range(4):
        lax.fori_loop(r * ch, (r + 1) * ch, extract_body, 0)
        if r >= 2:
            wr[r - 2].wait()
        gv = idx_v[pl.ds(r * ch, ch)] + b * s
        rd[r] = pltpu.async_copy(hs2_hbm.at[gv], bufs[r % 2], semr[r % 2])
        if r >= 1:
            rd[r - 1].wait()
            wr[r - 1] = pltpu.async_copy(
                bufs[(r - 1) % 2],
                gath_hbm.at[b, h, pl.ds((r - 1) * ch, ch)],
                semw[(r - 1) % 2])
    pltpu.sync_copy(maskl_v, part_hbm.at[b, h])
    rd[3].wait()
    wr[3] = pltpu.async_copy(
        bufs[3 % 2], gath_hbm.at[b, h, pl.ds(3 * ch, ch)], semw[3 % 2])
    wr[2].wait()
    wr[3].wait()


def _proj_kernel(g_ref, wp_ref, part_ref, o_ref, mask_ref):
    o_ref[0] = jax.lax.dot_general(
        g_ref[0, 0], wp_ref[0], (((1,), (1,)), ((), ())),
        preferred_element_type=jnp.float32)
    mask_ref[0] = jnp.max(part_ref[0], axis=0, keepdims=True)


def kernel(hidden_states, W_proj, W_gate):
    b, s, hidden = hidden_states.shape

    gates = pl.pallas_call(
        _gate_kernel,
        grid=(b, s // SBLK),
        in_specs=[
            pl.BlockSpec((1, SBLK, hidden), lambda i, j: (i, j, 0)),
            pl.BlockSpec((NH, hidden), lambda i, j: (0, 0)),
        ],
        out_specs=pl.BlockSpec((1, NH, SBLK), lambda i, j: (i, 0, j)),
        out_shape=jax.ShapeDtypeStruct((b, NH, s), jnp.float32),
    )(hidden_states, W_gate)

    part, gath = pl.kernel(
        _topk_sc_kernel,
        out_type=[
            jax.ShapeDtypeStruct((b, NH, s), jnp.float32),
            jax.ShapeDtypeStruct((b, NH, K, hidden), jnp.float32),
        ],
        mesh=plsc.VectorSubcoreMesh(core_axis_name="c", subcore_axis_name="s"),
        compiler_params=pltpu.CompilerParams(needs_layout_passes=False),
        scratch_types=[
            pltpu.VMEM((s,), jnp.float32),
            pltpu.VMEM((s // LANES,), jnp.float32),
            pltpu.VMEM((K,), jnp.int32),
            pltpu.VMEM((s,), jnp.float32),
            pltpu.VMEM((K // 4, hidden), jnp.float32),
            pltpu.VMEM((K // 4, hidden), jnp.float32),
            pltpu.SemaphoreType.DMA,
            pltpu.SemaphoreType.DMA,
            pltpu.SemaphoreType.DMA,
            pltpu.SemaphoreType.DMA,
        ],
    )(gates, hidden_states.reshape(b * s, hidden))

    wp3 = W_proj.reshape(NH, HD, hidden)
    out_states, maskf = pl.pallas_call(
        _proj_kernel,
        grid=(NH, b),
        in_specs=[
            pl.BlockSpec((1, 1, K, hidden), lambda j, i: (i, j, 0, 0)),
            pl.BlockSpec((1, HD, hidden), lambda j, i: (j, 0, 0)),
            pl.BlockSpec((1, NH, s), lambda j, i: (i, 0, 0)),
        ],
        out_specs=[
            pl.BlockSpec((1, K, HD), lambda j, i: (i, j, 0)),
            pl.BlockSpec((1, 1, s), lambda j, i: (i, 0, 0)),
        ],
        out_shape=[
            jax.ShapeDtypeStruct((b, NH * K, HD), jnp.float32),
            jax.ShapeDtypeStruct((b, 1, s), jnp.float32),
        ],
    )(gath, wp3, part)

    return out_states, maskf.reshape(b, s).astype(bool)
